# Initial kernel scaffold; baseline (speedup 1.0000x reference)
import functools

import jax
import jax.numpy as jnp
from jax import lax
from jax.experimental import pallas as pl
from jax.experimental.pallas import tpu as pltpu
from jax.experimental.pallas import tpu_sc as plsc

N = 3200000
NSEG = 100000
INC = 16
OUTC = 16
FEATC = 32

HALF = NSEG // 2
K = 512
RB = 1024
FU = 128
NTILE = 16
NCORE = 2

_mesh = plsc.VectorSubcoreMesh(core_axis_name="c", subcore_axis_name="s")


def _fc_body(x_ref, w_ref, b_ref, o_ref):
  h = lax.dot_general(x_ref[...], w_ref[...], (((1,), (1,)), ((), ())),
                      preferred_element_type=jnp.float32)
  o_ref[...] = h + b_ref[...]


def _compute_h(x, w_fc, b_fc):
  blk = 12800
  grid = (N // blk,)
  return pl.pallas_call(
      _fc_body,
      grid=grid,
      in_specs=[
          pl.BlockSpec((blk, INC), lambda i: (i, 0)),
          pl.BlockSpec((OUTC, INC), lambda i: (0, 0)),
          pl.BlockSpec((1, OUTC), lambda i: (0, 0)),
      ],
      out_specs=pl.BlockSpec((blk, OUTC), lambda i: (i, 0)),
      out_shape=jax.ShapeDtypeStruct((N, OUTC), jnp.float32),
  )(x, w_fc, b_fc)


def _sc1_body(h_hbm, ids_hbm, stats_hbm, cnts_hbm,
              ids_loc, h_loc, ring_d, ring_c, ring_s, probe,
              stats_sp, cnts_sp):
  c = lax.axis_index("c")
  s = lax.axis_index("s")
  zero16 = jnp.zeros((16,), jnp.float32)
  zero16i = jnp.zeros((16,), jnp.int32)

  def _zr(i, _):
    ring_d[i, 0:16] = zero16
    ring_d[i, 16:32] = zero16
    ring_c[i, 0] = jnp.int32(0)
    return 0
  lax.fori_loop(0, RB, _zr, 0)

  def _zs(i, _):
    for u in range(FU // 16):
      ring_s[i, pl.ds(u * 16, 16)] = zero16i
    return 0
  lax.fori_loop(0, RB // FU, _zs, 0)

  rows_per_tile = HALF // NTILE
  r0 = s * rows_per_tile
  for u in range(5):
    pltpu.sync_copy(ring_d.at[pl.ds(0, 625)],
                    stats_sp.at[pl.ds(r0 + u * 625, 625)])
    pltpu.sync_copy(ring_c.at[pl.ds(0, 625)],
                    cnts_sp.at[pl.ds(r0 + u * 625, 625)])
  plsc.subcore_barrier()

  def _bs(_, lohi):
    lo, hi = lohi
    mid = lax.div(lo + hi, jnp.int32(2))
    base = jnp.minimum(mid & ~7, N - 16)
    pltpu.sync_copy(ids_hbm.at[pl.ds(base, 16)], probe)
    v = probe[mid - base]
    take = v < HALF
    return (jnp.where(take, mid + 1, lo), jnp.where(take, hi, mid))
  nlo, _ = lax.fori_loop(0, 22, _bs, (jnp.int32(0), jnp.int32(N)))

  p0 = jnp.where(c == 0, jnp.int32(0), nlo)
  p1 = jnp.where(c == 0, nlo, jnp.int32(N))
  q = lax.div(p1 - p0 + NTILE - 1, jnp.int32(NTILE))
  lo_t = jnp.minimum(p0 + s * q, p1)
  hi_t = jnp.minimum(lo_t + q, p1)
  seg_base = c * HALF

  def _emit(carry):
    cur_seg, cnt, rank, flushed, acc, acc2 = carry
    j = rank & (RB - 1)
    ring_d[j, 0:16] = acc
    ring_d[j, 16:32] = acc2
    ring_c[j, 0] = cnt
    lid = jnp.clip(cur_seg - seg_base, 0, HALF - 1)
    ring_s[j >> 7, j & (FU - 1)] = lid
    return rank + 1

  def _flush(rf):
    rank, flushed = rf
    fb = flushed & (RB - 1)
    fr = fb >> 7
    pltpu.sync_copy(ring_d.at[pl.ds(fb, FU)],
                    stats_sp.at[ring_s.at[fr]], add=True)
    pltpu.sync_copy(ring_c.at[pl.ds(fb, FU)],
                    cnts_sp.at[ring_s.at[fr]], add=True)

    def _rz(i, _):
      ring_d[i, 0:16] = zero16
      ring_d[i, 16:32] = zero16
      ring_c[i, 0] = jnp.int32(0)
      return 0
    lax.fori_loop(fb, fb + FU, _rz, 0)
    return (rank, flushed + FU)

  def _chunk(cj, carry):
    base = cj * K
    pltpu.sync_copy(ids_hbm.at[pl.ds(base, K)], ids_loc)
    pltpu.sync_copy(h_hbm.at[pl.ds(base, K)], h_loc)
    i0 = jnp.maximum(lo_t - base, 0)
    i1 = jnp.minimum(hi_t - base, K)

    def _point(i, pc):
      sid = ids_loc[i]
      row = h_loc[i]

      def _new(pc):
        rank = _emit(pc)
        return (sid, jnp.int32(1), rank, pc[3], row, row * row)

      def _same(pc):
        cur_seg, cnt, rank, flushed, acc, acc2 = pc
        return (cur_seg, cnt + 1, rank, flushed, acc + row,
                acc2 + row * row)

      return lax.cond(sid != pc[0], _new, _same, pc)

    carry = lax.fori_loop(i0, i1, _point, carry)
    cur_seg, cnt, rank, flushed, acc, acc2 = carry
    rank, flushed = lax.while_loop(
        lambda rf: rf[0] - rf[1] >= FU, _flush, (rank, flushed))
    return (cur_seg, cnt, rank, flushed, acc, acc2)

  init = (jnp.int32(-1), jnp.int32(0), jnp.int32(0), jnp.int32(0),
          zero16, zero16)
  carry = lax.fori_loop(lax.div(lo_t, jnp.int32(K)),
                        lax.div(hi_t + K - 1, jnp.int32(K)),
                        _chunk, init)
  rank = _emit(carry)
  flushed = carry[3]
  rank, flushed = lax.while_loop(
      lambda rf: rf[1] < rf[0], _flush, (rank, flushed))

  plsc.subcore_barrier()
  pltpu.sync_copy(stats_sp.at[pl.ds(r0, rows_per_tile)],
                  stats_hbm.at[pl.ds(seg_base + r0, rows_per_tile)])
  pltpu.sync_copy(cnts_sp.at[pl.ds(r0, rows_per_tile)],
                  cnts_hbm.at[pl.ds(seg_base + r0, rows_per_tile)])


_sc1 = functools.partial(
    pl.kernel,
    out_type=(jax.ShapeDtypeStruct((NSEG, 32), jnp.float32),
              jax.ShapeDtypeStruct((NSEG, 8), jnp.int32)),
    mesh=_mesh,
    scratch_types=[
        pltpu.VMEM((K,), jnp.int32),
        pltpu.VMEM((K, 16), jnp.float32),
        pltpu.VMEM((RB, 32), jnp.float32),
        pltpu.VMEM((RB, 8), jnp.int32),
        pltpu.VMEM((RB // FU, FU), jnp.int32),
        pltpu.VMEM((16,), jnp.int32),
        pltpu.VMEM_SHARED((HALF, 32), jnp.float32),
        pltpu.VMEM_SHARED((HALF, 8), jnp.int32),
    ],
)


def _run_sc1(h, ids32):
  return _sc1(_sc1_body)(h, ids32)


def _ab_body(st_ref, ct_ref, of_ref, wmu_ref, bmu_ref, wsig_ref, bsig_ref,
             ab_ref):
  sums = st_ref[:, 0:16]
  sqs = st_ref[:, 16:32]
  cnt = jnp.maximum(ct_ref[:, 0:1].astype(jnp.float32), 1.0)
  mean = sums / cnt
  var = jnp.maximum(sqs / cnt - mean * mean, 0.0)
  rstd = lax.rsqrt(var + 1e-14)
  of = of_ref[...]
  mu = lax.dot_general(of, wmu_ref[...], (((1,), (1,)), ((), ())),
                       preferred_element_type=jnp.float32) + bmu_ref[...]
  sig = lax.dot_general(of, wsig_ref[...], (((1,), (1,)), ((), ())),
                        preferred_element_type=jnp.float32) + bsig_ref[...]
  a = sig * rstd
  b = mu - mean * a
  ab_ref[:, 0:16] = a
  ab_ref[:, 16:32] = b


def _compute_ab(stats, cnts, origin_feat, w_mu, b_mu, w_sig, b_sig):
  blk = 4000
  grid = (NSEG // blk,)
  return pl.pallas_call(
      _ab_body,
      grid=grid,
      in_specs=[
          pl.BlockSpec((blk, 32), lambda i: (i, 0)),
          pl.BlockSpec((blk, 8), lambda i: (i, 0)),
          pl.BlockSpec((blk, FEATC), lambda i: (i, 0)),
          pl.BlockSpec((OUTC, FEATC), lambda i: (0, 0)),
          pl.BlockSpec((1, OUTC), lambda i: (0, 0)),
          pl.BlockSpec((OUTC, FEATC), lambda i: (0, 0)),
          pl.BlockSpec((1, OUTC), lambda i: (0, 0)),
      ],
      out_specs=pl.BlockSpec((blk, 32), lambda i: (i, 0)),
      out_shape=jax.ShapeDtypeStruct((NSEG, 32), jnp.float32),
  )(stats, cnts, origin_feat, w_mu, b_mu, w_sig, b_sig)


NCHUNK = N // K
NW = NCORE * NTILE


def _sc2_body(h_hbm, ids2d_hbm, ab_hbm, out_hbm,
              idsb, h_loc, ab_loc, out_loc, sem):
  c = lax.axis_index("c")
  s = lax.axis_index("s")
  wid = s * NCORE + c

  def _chunk(t, _):
    cj = t * NW + wid

    @pl.when(cj < NCHUNK)
    def _():
      base = cj * K
      pltpu.sync_copy(ids2d_hbm.at[pl.ds(cj * (K // FU), K // FU)], idsb)
      pltpu.sync_copy(h_hbm.at[pl.ds(base, K)], h_loc)
      for m in range(K // FU):
        pltpu.async_copy(ab_hbm.at[idsb.at[m]],
                         ab_loc.at[pl.ds(m * FU, FU)], sem).wait()

      def _pts(p8, _):
        for u in range(8):
          p = p8 * 8 + u
          row = h_loc[p]
          a = ab_loc[p, 0:16]
          b = ab_loc[p, 16:32]
          out_loc[p] = jnp.maximum(row * a + b, 0.0)
        return 0
      lax.fori_loop(0, K // 8, _pts, 0)
      pltpu.sync_copy(out_loc, out_hbm.at[pl.ds(base, K)])
    return 0

  nt = (NCHUNK + NW - 1) // NW
  lax.fori_loop(0, nt, _chunk, 0)


_sc2 = functools.partial(
    pl.kernel,
    out_type=jax.ShapeDtypeStruct((N, OUTC), jnp.float32),
    mesh=_mesh,
    scratch_types=[
        pltpu.VMEM((K // FU, FU), jnp.int32),
        pltpu.VMEM((K, 16), jnp.float32),
        pltpu.VMEM((K, 32), jnp.float32),
        pltpu.VMEM((K, 16), jnp.float32),
        pltpu.SemaphoreType.DMA,
    ],
)


def _run_sc2(h, ids2d, ab):
  return _sc2(_sc2_body)(h, ids2d, ab)


def kernel(x, origin_feat, latent_point_batch, W_fc, b_fc, W_mu, b_mu,
           W_sig, b_sig):
  ids32 = latent_point_batch.astype(jnp.int32)
  ids2d = ids32.reshape(N // FU, FU)
  h = _compute_h(x, W_fc, b_fc.reshape(1, OUTC))
  stats, cnts = _run_sc1(h, ids32)
  ab = _compute_ab(stats, cnts, origin_feat, W_mu, b_mu.reshape(1, OUTC),
                   W_sig, b_sig.reshape(1, OUTC))
  return _run_sc2(h, ids2d, ab)


# speedup vs baseline: 9.7993x; 9.7993x over previous
import functools

import jax
import jax.numpy as jnp
from jax import lax
from jax.experimental import pallas as pl
from jax.experimental.pallas import tpu as pltpu
from jax.experimental.pallas import tpu_sc as plsc

N = 3200000
NSEG = 100000
INC = 16
OUTC = 16
FEATC = 32

NTILE = 16
NCORE = 2
K = 512

_mesh = plsc.VectorSubcoreMesh(core_axis_name="c", subcore_axis_name="s")


def _fc_body(x_ref, wbd_ref, bbd_ref, ids_ref, o_ref, sp_ref):
  h = lax.dot_general(x_ref[...], wbd_ref[...], (((1,), (0,)), ((), ())),
                      preferred_element_type=jnp.float32)
  o_ref[...] = h + bbd_ref[...]

  @pl.when(pl.program_id(0) == 0)
  def _():
    sp_ref[...] = jnp.zeros((1, 128), jnp.int32)

  ids = ids_ref[...]
  lane = lax.broadcasted_iota(jnp.int32, (1, 128), 1)
  vec = jnp.zeros((1, 128), jnp.int32)
  for k in range(1, 32):
    ck = jnp.sum((ids < k * 3136).astype(jnp.int32))
    vec = vec + jnp.where(lane == k - 1, ck, 0)
  sp_ref[...] += vec


def _compute_h(x2d, w_bd, b_bd, ids3d_tc):
  blk = 12800
  grid = (N // blk,)
  return pl.pallas_call(
      _fc_body,
      grid=grid,
      in_specs=[
          pl.BlockSpec((blk // 8, 128), lambda i: (i, 0)),
          pl.BlockSpec((128, 128), lambda i: (0, 0)),
          pl.BlockSpec((1, 128), lambda i: (0, 0)),
          pl.BlockSpec((1, 1, blk), lambda i: (i, 0, 0)),
      ],
      out_specs=[
          pl.BlockSpec((blk // 8, 128), lambda i: (i, 0)),
          pl.BlockSpec((1, 128), lambda i: (0, 0)),
      ],
      out_shape=(jax.ShapeDtypeStruct((N // 8, 128), jnp.float32),
                 jax.ShapeDtypeStruct((1, 128), jnp.int32)),
  )(x2d, w_bd, b_bd, ids3d_tc)


SPAN = 3136
NSEGP = 32 * SPAN


def _sc1_body(h_hbm, ids_hbm, sp_hbm, stats_hbm, cnts_hbm,
              ids_loc, h_loc, probe, win_d, cnt_win, acc_v, acc2_v):
  c = lax.axis_index("c")
  s = lax.axis_index("s")
  wid = c * NTILE + s
  zero16 = jnp.zeros((16,), jnp.float32)
  zero16i = jnp.zeros((16,), jnp.int32)
  iota16 = lax.iota(jnp.int32, 16)

  def _lane_x(v, lane):
    idx = jnp.where(iota16 == 0, lane, 0)
    return jnp.take_along_axis(v, idx, axis=0)[0]

  def _zw(i, _):
    for u in range(8):
      win_d[i, pl.ds(u * 16, 16)] = zero16
    return 0
  lax.fori_loop(0, SPAN // 4, _zw, 0)

  def _zc(i, _):
    cnt_win[pl.ds(i * 16, 16)] = zero16i
    return 0
  lax.fori_loop(0, SPAN // 16, _zc, 0)

  pltpu.sync_copy(sp_hbm.at[pl.ds(0, 32)], probe)
  v0 = probe[0:16]
  v1 = probe[16:32]

  def _bound(k):
    km = k - 1
    lo = _lane_x(v0, jnp.clip(km, 0, 15))
    hi = _lane_x(v1, jnp.clip(km - 16, 0, 15))
    return jnp.where(k == 0, 0,
                     jnp.where(k >= 32, N, jnp.where(km < 16, lo, hi)))

  lo_t = _bound(wid)
  hi_t = _bound(wid + 1)
  seg_lo = wid * SPAN

  def _emit(cur_seg, cnt):
    lid = jnp.clip(cur_seg - seg_lo, 0, SPAN - 1)
    r = lid >> 2
    col = pl.multiple_of((lid & 3) * 32, 32)
    win_d[r, pl.ds(col, 16)] = acc_v[...]
    win_d[r, pl.ds(col + 16, 16)] = acc2_v[...]
    cb = pl.multiple_of((lid >> 4) << 4, 16)
    cv = cnt_win[pl.ds(cb, 16)]
    cnt_win[pl.ds(cb, 16)] = jnp.where(iota16 == lid - cb, cnt, cv)

  def _step(sid, row, pc):
    def _new(pc):
      cur_seg, cnt = pc
      _emit(cur_seg, cnt)
      acc_v[...] = row
      acc2_v[...] = row * row
      return (sid, jnp.int32(1))

    def _same(pc):
      cur_seg, cnt = pc
      plsc.addupdate(acc_v.at[...], row)
      plsc.addupdate(acc2_v.at[...], row * row)
      return (sid, cnt + 1)

    return lax.cond(sid != pc[0], _new, _same, pc)

  def _point(i, pc):
    g = pl.multiple_of((i >> 4) << 4, 16)
    sv = ids_loc[pl.ds(g, 16)]
    sid = _lane_x(sv, i - g)
    col = pl.multiple_of((i & 7) * 16, 16)
    return _step(sid, h_loc[i >> 3, pl.ds(col, 16)], pc)

  def _group(gi, pc):
    g = gi * 16
    sv = ids_loc[pl.ds(g, 16)]
    rw = g >> 3

    def _fast(pc):
      rows = [h_loc[rw + (u >> 3), (u & 7) * 16:(u & 7) * 16 + 16]
              for u in range(16)]
      sm = rows[0]
      sq = rows[0] * rows[0]
      for u in range(1, 16):
        sm = sm + rows[u]
        sq = sq + rows[u] * rows[u]

      def _cont(pc):
        cur_seg, cnt = pc
        plsc.addupdate(acc_v.at[...], sm)
        plsc.addupdate(acc2_v.at[...], sq)
        return (cur_seg, cnt + 16)

      def _brk(pc):
        cur_seg, cnt = pc
        _emit(cur_seg, cnt)
        acc_v[...] = sm
        acc2_v[...] = sq
        return (sv[0], jnp.int32(16))

      return lax.cond(sv[0] == pc[0], _cont, _brk, pc)

    def _slow(pc):
      for u in range(16):
        pc = _step(sv[u],
                   h_loc[rw + (u >> 3), (u & 7) * 16:(u & 7) * 16 + 16],
                   pc)
      return pc

    return lax.cond(sv[0] == sv[15], _fast, _slow, pc)

  def _chunk(cj, carry):
    base = pl.multiple_of(cj * K, K)
    pltpu.sync_copy(ids_hbm.at[pl.ds(base, K)], ids_loc)
    pltpu.sync_copy(h_hbm.at[pl.ds(pl.multiple_of(cj * (K // 8), K // 8),
                                   K // 8)], h_loc)
    i0 = jnp.maximum(lo_t - base, 0)
    i1 = jnp.minimum(hi_t - base, K)
    a = jnp.minimum((i0 + 15) & ~15, i1)
    b = jnp.maximum(i1 & ~15, a)
    carry = lax.fori_loop(i0, a, _point, carry)
    carry = lax.fori_loop(a >> 4, b >> 4, _group, carry)
    carry = lax.fori_loop(b, i1, _point, carry)
    return carry

  acc_v[...] = zero16
  acc2_v[...] = zero16
  init = (jnp.int32(-1), jnp.int32(0))
  cur_seg, cnt = lax.fori_loop(lax.div(lo_t, jnp.int32(K)),
                               lax.div(hi_t + K - 1, jnp.int32(K)),
                               _chunk, init)
  _emit(jnp.where(cnt > 0, cur_seg, seg_lo), cnt)

  pltpu.sync_copy(win_d, stats_hbm.at[pl.ds(wid * (SPAN // 4), SPAN // 4)])
  pltpu.sync_copy(cnt_win, cnts_hbm.at[pl.ds(wid * SPAN, SPAN)])


_sc1 = functools.partial(
    pl.kernel,
    out_type=(jax.ShapeDtypeStruct((NSEGP // 4, 128), jnp.float32),
              jax.ShapeDtypeStruct((NSEGP,), jnp.int32)),
    mesh=_mesh,
    scratch_types=[
        pltpu.VMEM((K,), jnp.int32),
        pltpu.VMEM((K // 8, 128), jnp.float32),
        pltpu.VMEM((32,), jnp.int32),
        pltpu.VMEM((SPAN // 4, 128), jnp.float32),
        pltpu.VMEM((SPAN,), jnp.int32),
        pltpu.VMEM((16,), jnp.float32),
        pltpu.VMEM((16,), jnp.float32),
    ],
)


def _run_sc1(h2d, ids32, splits):
  return _sc1(_sc1_body)(h2d, ids32, splits)


def _ab_body(st_ref, ct_ref, of_ref, wmu_ref, bmu_ref, wsig_ref, bsig_ref,
             ab_ref):
  sums = st_ref[:, 0:16]
  sqs = st_ref[:, 16:32]
  cnt = jnp.maximum(ct_ref[...].astype(jnp.float32), 1.0)
  mean = sums / cnt
  var = jnp.maximum(sqs / cnt - mean * mean, 0.0)
  rstd = lax.rsqrt(var + 1e-14)
  of = of_ref[...]
  mu = lax.dot_general(of, wmu_ref[...], (((1,), (1,)), ((), ())),
                       preferred_element_type=jnp.float32) + bmu_ref[...]
  sig = lax.dot_general(of, wsig_ref[...], (((1,), (1,)), ((), ())),
                        preferred_element_type=jnp.float32) + bsig_ref[...]
  a = sig * rstd
  b = mu - mean * a
  ab_ref[:, 0:16] = a
  ab_ref[:, 16:32] = b


def _compute_ab(stats, cnts, origin_feat, w_mu, b_mu, w_sig, b_sig):
  blk = 4000
  grid = (NSEG // blk,)
  return pl.pallas_call(
      _ab_body,
      grid=grid,
      in_specs=[
          pl.BlockSpec((blk, 32), lambda i: (i, 0)),
          pl.BlockSpec((blk, 1), lambda i: (i, 0)),
          pl.BlockSpec((blk, FEATC), lambda i: (i, 0)),
          pl.BlockSpec((OUTC, FEATC), lambda i: (0, 0)),
          pl.BlockSpec((1, OUTC), lambda i: (0, 0)),
          pl.BlockSpec((OUTC, FEATC), lambda i: (0, 0)),
          pl.BlockSpec((1, OUTC), lambda i: (0, 0)),
      ],
      out_specs=pl.BlockSpec((blk, 32), lambda i: (i, 0)),
      out_shape=jax.ShapeDtypeStruct((NSEG, 32), jnp.float32),
  )(stats, cnts, origin_feat, w_mu, b_mu, w_sig, b_sig)


NCHUNK = N // K
NW = NCORE * NTILE
WAB = 256


def _sc2_body(h_hbm, ids_hbm, ab_hbm, out_hbm,
              ids_loc, h_loc, ab_win, out_loc):
  c = lax.axis_index("c")
  s = lax.axis_index("s")
  wid = s * NCORE + c
  iota16 = lax.iota(jnp.int32, 16)

  def _reload(sid, win):
    nw = pl.multiple_of(jnp.minimum(sid & ~7, NSEG - WAB), 8)
    pltpu.sync_copy(ab_hbm.at[pl.ds(nw, WAB)], ab_win)
    return nw

  def _apply(pt, sid, win):
    col = pl.multiple_of((pt & 7) * 16, 16)
    row = h_loc[pt >> 3, pl.ds(col, 16)]
    lid = sid - win
    a = ab_win[lid, 0:16]
    b = ab_win[lid, 16:32]
    out_loc[pt >> 3, pl.ds(col, 16)] = jnp.maximum(row * a + b, 0.0)

  def _chunk(t, win):
    cj = t * NW + wid

    def _do(win):
      base = pl.multiple_of(cj * K, K)
      pltpu.sync_copy(ids_hbm.at[pl.ds(base, K)], ids_loc)
      pltpu.sync_copy(h_hbm.at[pl.ds(pl.multiple_of(cj * (K // 8), K // 8),
                                     K // 8)], h_loc)

      def _group(gi, win):
        g = gi * 16
        sv = ids_loc[pl.ds(g, 16)]

        def _gfast(win):
          sid = sv[0]
          win = lax.cond(sid - win >= WAB,
                         lambda w: _reload(sid, w), lambda w: w, win)
          lid = sid - win
          a = ab_win[lid, 0:16]
          b = ab_win[lid, 16:32]
          rw = g >> 3
          for u in range(16):
            cs = (u & 7) * 16
            row = h_loc[rw + (u >> 3), cs:cs + 16]
            out_loc[rw + (u >> 3), cs:cs + 16] = jnp.maximum(row * a + b, 0.0)
          return win

        def _gslow(win):
          for u in range(16):
            sid = sv[u]
            win = lax.cond(sid - win >= WAB,
                           lambda w: _reload(sid, w), lambda w: w, win)
            _apply(g + u, sid, win)
          return win

        return lax.cond(sv[0] == sv[15], _gfast, _gslow, win)

      win = lax.fori_loop(0, K // 16, _group, win)
      pltpu.sync_copy(out_loc,
                      out_hbm.at[pl.ds(pl.multiple_of(cj * (K // 8), K // 8),
                                       K // 8)])
      return win

    return lax.cond(cj < NCHUNK, _do, lambda w: w, win)

  lax.fori_loop(0, (NCHUNK + NW - 1) // NW, _chunk, jnp.int32(-4 * WAB))


_sc2 = functools.partial(
    pl.kernel,
    out_type=jax.ShapeDtypeStruct((N // 8, 128), jnp.float32),
    mesh=_mesh,
    scratch_types=[
        pltpu.VMEM((K,), jnp.int32),
        pltpu.VMEM((K // 8, 128), jnp.float32),
        pltpu.VMEM((WAB, 32), jnp.float32),
        pltpu.VMEM((K // 8, 128), jnp.float32),
    ],
)


def _run_sc2(h2d, ids32, ab):
  return _sc2(_sc2_body)(h2d, ids32, ab)


def kernel(x, origin_feat, latent_point_batch, W_fc, b_fc, W_mu, b_mu,
           W_sig, b_sig):
  ids32 = latent_point_batch.astype(jnp.int32)
  ids3d_tc = ids32.reshape(N // 12800, 1, 12800)
  x2d = x.reshape(N // 8, 128)
  bi = jnp.arange(128)[:, None] // 16
  bj = jnp.arange(128)[None, :] // 16
  w_bd = jnp.where(bi == bj, jnp.tile(W_fc.T, (8, 8)), 0.0).astype(jnp.float32)
  b_bd = jnp.tile(b_fc, 8).reshape(1, 128)
  h2d, splits = _compute_h(x2d, w_bd, b_bd, ids3d_tc)
  stats2d, cnts = _run_sc1(h2d, ids32, splits.reshape(128))
  stats = stats2d.reshape(NSEGP, 32)[:NSEG]
  ab = _compute_ab(stats, cnts[:NSEG].reshape(NSEG, 1), origin_feat,
                   W_mu, b_mu.reshape(1, OUTC), W_sig, b_sig.reshape(1, OUTC))
  return _run_sc2(h2d, ids32, ab).reshape(N, OUTC)


# speedup vs baseline: 10.6657x; 1.0884x over previous
import functools

import jax
import jax.numpy as jnp
from jax import lax
from jax.experimental import pallas as pl
from jax.experimental.pallas import tpu as pltpu
from jax.experimental.pallas import tpu_sc as plsc

N = 3200000
NSEG = 100000
INC = 16
OUTC = 16
FEATC = 32

NTILE = 16
NCORE = 2
K = 512

_mesh = plsc.VectorSubcoreMesh(core_axis_name="c", subcore_axis_name="s")


def _fc_body(x_ref, wbd_ref, bbd_ref, ids_ref, o_ref, sp_ref):
  h = lax.dot_general(x_ref[...], wbd_ref[...], (((1,), (0,)), ((), ())),
                      preferred_element_type=jnp.float32)
  o_ref[...] = h + bbd_ref[...]

  @pl.when(pl.program_id(0) == 0)
  def _():
    sp_ref[...] = jnp.zeros((1, 128), jnp.int32)

  ids = ids_ref[...]
  lane = lax.broadcasted_iota(jnp.int32, (1, 128), 1)
  vec = jnp.zeros((1, 128), jnp.int32)
  for k in range(1, 32):
    ck = jnp.sum((ids < k * 3136).astype(jnp.int32))
    vec = vec + jnp.where(lane == k - 1, ck, 0)
  sp_ref[...] += vec


def _compute_h(x2d, w_bd, b_bd, ids3d_tc):
  blk = 12800
  grid = (N // blk,)
  return pl.pallas_call(
      _fc_body,
      grid=grid,
      in_specs=[
          pl.BlockSpec((blk // 8, 128), lambda i: (i, 0)),
          pl.BlockSpec((128, 128), lambda i: (0, 0)),
          pl.BlockSpec((1, 128), lambda i: (0, 0)),
          pl.BlockSpec((1, 1, blk), lambda i: (i, 0, 0)),
      ],
      out_specs=[
          pl.BlockSpec((blk // 8, 128), lambda i: (i, 0)),
          pl.BlockSpec((1, 128), lambda i: (0, 0)),
      ],
      out_shape=(jax.ShapeDtypeStruct((N // 8, 128), jnp.float32),
                 jax.ShapeDtypeStruct((1, 128), jnp.int32)),
  )(x2d, w_bd, b_bd, ids3d_tc)


SPAN = 3136
NSEGP = 32 * SPAN


def _sc1_body(h_hbm, ids_hbm, sp_hbm, stats_hbm, cnts_hbm,
              ids_loc, h_loc, probe, win_d, cnt_win, acc_v, acc2_v, sem):
  c = lax.axis_index("c")
  s = lax.axis_index("s")
  wid = c * NTILE + s
  zero16 = jnp.zeros((16,), jnp.float32)
  zero16i = jnp.zeros((16,), jnp.int32)
  iota16 = lax.iota(jnp.int32, 16)

  def _lane_x(v, lane):
    idx = jnp.where(iota16 == 0, lane, 0)
    return jnp.take_along_axis(v, idx, axis=0)[0]

  def _zw(i, _):
    for u in range(8):
      win_d[i, pl.ds(u * 16, 16)] = zero16
    return 0
  lax.fori_loop(0, SPAN // 4, _zw, 0)

  def _zc(i, _):
    cnt_win[pl.ds(i * 16, 16)] = zero16i
    return 0
  lax.fori_loop(0, SPAN // 16, _zc, 0)

  pltpu.sync_copy(sp_hbm.at[pl.ds(0, 32)], probe)
  v0 = probe[0:16]
  v1 = probe[16:32]

  def _bound(k):
    km = k - 1
    lo = _lane_x(v0, jnp.clip(km, 0, 15))
    hi = _lane_x(v1, jnp.clip(km - 16, 0, 15))
    return jnp.where(k == 0, 0,
                     jnp.where(k >= 32, N, jnp.where(km < 16, lo, hi)))

  lo_t = _bound(wid)
  hi_t = _bound(wid + 1)
  seg_lo = wid * SPAN

  def _emit(cur_seg, cnt):
    lid = jnp.clip(cur_seg - seg_lo, 0, SPAN - 1)
    r = lid >> 2
    col = pl.multiple_of((lid & 3) * 32, 32)
    win_d[r, pl.ds(col, 16)] = acc_v[...]
    win_d[r, pl.ds(col + 16, 16)] = acc2_v[...]
    cb = pl.multiple_of((lid >> 4) << 4, 16)
    cv = cnt_win[pl.ds(cb, 16)]
    cnt_win[pl.ds(cb, 16)] = jnp.where(iota16 == lid - cb, cnt, cv)

  def _step(sid, row, pc):
    def _new(pc):
      cur_seg, cnt = pc
      _emit(cur_seg, cnt)
      acc_v[...] = row
      acc2_v[...] = row * row
      return (sid, jnp.int32(1))

    def _same(pc):
      cur_seg, cnt = pc
      plsc.addupdate(acc_v.at[...], row)
      plsc.addupdate(acc2_v.at[...], row * row)
      return (sid, cnt + 1)

    return lax.cond(sid != pc[0], _new, _same, pc)

  def _point(i, o, ro, pc):
    g = pl.multiple_of((i >> 4) << 4, 16)
    sv = ids_loc[pl.ds(g + o, 16)]
    sid = _lane_x(sv, i - g)
    col = pl.multiple_of((i & 7) * 16, 16)
    return _step(sid, h_loc[ro + (i >> 3), pl.ds(col, 16)], pc)

  def _group(gi, o, ro, pc):
    g = gi * 16
    sv = ids_loc[pl.ds(g + o, 16)]
    rw = (g >> 3) + ro

    def _fast(pc):
      rows = [h_loc[rw + (u >> 3), (u & 7) * 16:(u & 7) * 16 + 16]
              for u in range(16)]
      sm = rows[0]
      sq = rows[0] * rows[0]
      for u in range(1, 16):
        sm = sm + rows[u]
        sq = sq + rows[u] * rows[u]

      def _cont(pc):
        cur_seg, cnt = pc
        plsc.addupdate(acc_v.at[...], sm)
        plsc.addupdate(acc2_v.at[...], sq)
        return (cur_seg, cnt + 16)

      def _brk(pc):
        cur_seg, cnt = pc
        _emit(cur_seg, cnt)
        acc_v[...] = sm
        acc2_v[...] = sq
        return (sv[0], jnp.int32(16))

      return lax.cond(sv[0] == pc[0], _cont, _brk, pc)

    def _slow(pc):
      for u in range(16):
        pc = _step(sv[u],
                   h_loc[rw + (u >> 3), (u & 7) * 16:(u & 7) * 16 + 16],
                   pc)
      return pc

    return lax.cond(sv[0] == sv[15], _fast, _slow, pc)

  def _io(cj, b):
    base = pl.multiple_of(cj * K, K)
    ic = pltpu.make_async_copy(ids_hbm.at[pl.ds(base, K)],
                               ids_loc.at[pl.ds(b * K, K)], sem)
    hc = pltpu.make_async_copy(
        h_hbm.at[pl.ds(pl.multiple_of(cj * (K // 8), K // 8), K // 8)],
        h_loc.at[pl.ds(b * (K // 8), K // 8)], sem)
    return ic, hc

  def _issue(cj, b):
    ic, hc = _io(cj, b)
    ic.start()
    hc.start()

  def _proc(cj, b, carry):
    ic, hc = _io(cj, b)
    ic.wait()
    hc.wait()

    @pl.when(cj + 1 < nc1)
    def _():
      _issue(cj + 1, 1 - b)

    base = pl.multiple_of(cj * K, K)
    o = b * K
    ro = b * (K // 8)
    i0 = jnp.maximum(lo_t - base, 0)
    i1 = jnp.minimum(hi_t - base, K)
    a = jnp.minimum((i0 + 15) & ~15, i1)
    bb = jnp.maximum(i1 & ~15, a)
    carry = lax.fori_loop(i0, a, lambda i, pc: _point(i, o, ro, pc), carry)
    carry = lax.fori_loop(a >> 4, bb >> 4,
                          lambda gi, pc: _group(gi, o, ro, pc), carry)
    carry = lax.fori_loop(bb, i1, lambda i, pc: _point(i, o, ro, pc), carry)
    return carry

  def _pair(t, carry):
    for b in range(2):
      cj = nc0 + 2 * t + b
      carry = lax.cond(cj < nc1,
                       lambda pc, cj=cj, b=b: _proc(cj, b, pc),
                       lambda pc: pc, carry)
    return carry

  acc_v[...] = zero16
  acc2_v[...] = zero16
  init = (jnp.int32(-1), jnp.int32(0))
  nc0 = lax.div(lo_t, jnp.int32(K))
  nc1 = lax.div(hi_t + K - 1, jnp.int32(K))

  @pl.when(nc1 > nc0)
  def _():
    _issue(nc0, 0)
  cur_seg, cnt = lax.fori_loop(0, lax.div(nc1 - nc0 + 1, jnp.int32(2)),
                               _pair, init)
  _emit(jnp.where(cnt > 0, cur_seg, seg_lo), cnt)

  pltpu.sync_copy(win_d, stats_hbm.at[pl.ds(wid * (SPAN // 4), SPAN // 4)])
  pltpu.sync_copy(cnt_win, cnts_hbm.at[pl.ds(wid * SPAN, SPAN)])


_sc1 = functools.partial(
    pl.kernel,
    out_type=(jax.ShapeDtypeStruct((NSEGP // 4, 128), jnp.float32),
              jax.ShapeDtypeStruct((NSEGP,), jnp.int32)),
    mesh=_mesh,
    scratch_types=[
        pltpu.VMEM((2 * K,), jnp.int32),
        pltpu.VMEM((2 * (K // 8), 128), jnp.float32),
        pltpu.VMEM((32,), jnp.int32),
        pltpu.VMEM((SPAN // 4, 128), jnp.float32),
        pltpu.VMEM((SPAN,), jnp.int32),
        pltpu.VMEM((16,), jnp.float32),
        pltpu.VMEM((16,), jnp.float32),
        pltpu.SemaphoreType.DMA,
    ],
)


def _run_sc1(h2d, ids32, splits):
  return _sc1(_sc1_body)(h2d, ids32, splits)


def _ab_body(st_ref, ct_ref, of_ref, wmu_ref, bmu_ref, wsig_ref, bsig_ref,
             ab_ref):
  sums = st_ref[:, 0:16]
  sqs = st_ref[:, 16:32]
  cnt = jnp.maximum(ct_ref[...].astype(jnp.float32), 1.0)
  mean = sums / cnt
  var = jnp.maximum(sqs / cnt - mean * mean, 0.0)
  rstd = lax.rsqrt(var + 1e-14)
  of = of_ref[...]
  mu = lax.dot_general(of, wmu_ref[...], (((1,), (1,)), ((), ())),
                       preferred_element_type=jnp.float32) + bmu_ref[...]
  sig = lax.dot_general(of, wsig_ref[...], (((1,), (1,)), ((), ())),
                        preferred_element_type=jnp.float32) + bsig_ref[...]
  a = sig * rstd
  b = mu - mean * a
  ab_ref[:, 0:16] = a
  ab_ref[:, 16:32] = b


def _compute_ab(stats, cnts, origin_feat, w_mu, b_mu, w_sig, b_sig):
  blk = 4000
  grid = (NSEG // blk,)
  return pl.pallas_call(
      _ab_body,
      grid=grid,
      in_specs=[
          pl.BlockSpec((blk, 32), lambda i: (i, 0)),
          pl.BlockSpec((blk, 1), lambda i: (i, 0)),
          pl.BlockSpec((blk, FEATC), lambda i: (i, 0)),
          pl.BlockSpec((OUTC, FEATC), lambda i: (0, 0)),
          pl.BlockSpec((1, OUTC), lambda i: (0, 0)),
          pl.BlockSpec((OUTC, FEATC), lambda i: (0, 0)),
          pl.BlockSpec((1, OUTC), lambda i: (0, 0)),
      ],
      out_specs=pl.BlockSpec((blk, 32), lambda i: (i, 0)),
      out_shape=jax.ShapeDtypeStruct((NSEG, 32), jnp.float32),
  )(stats, cnts, origin_feat, w_mu, b_mu, w_sig, b_sig)


NCHUNK = N // K
NW = NCORE * NTILE
WAB = 256


def _sc2_body(h_hbm, ids_hbm, ab_hbm, out_hbm,
              ids_loc, h_loc, ab_win, out_loc, sem, osem):
  c = lax.axis_index("c")
  s = lax.axis_index("s")
  wid = s * NCORE + c
  iota16 = lax.iota(jnp.int32, 16)

  def _reload(sid, win):
    nw = pl.multiple_of(jnp.minimum(sid & ~7, NSEG - WAB), 8)
    pltpu.sync_copy(ab_hbm.at[pl.ds(nw, WAB)], ab_win)
    return nw

  def _apply(pt, sid, ro, win):
    col = pl.multiple_of((pt & 7) * 16, 16)
    row = h_loc[ro + (pt >> 3), pl.ds(col, 16)]
    lid = sid - win
    a = ab_win[lid, 0:16]
    b = ab_win[lid, 16:32]
    out_loc[pt >> 3, pl.ds(col, 16)] = jnp.maximum(row * a + b, 0.0)

  def _io(cj, b):
    base = pl.multiple_of(cj * K, K)
    ic = pltpu.make_async_copy(ids_hbm.at[pl.ds(base, K)],
                               ids_loc.at[pl.ds(b * K, K)], sem)
    hc = pltpu.make_async_copy(
        h_hbm.at[pl.ds(pl.multiple_of(cj * (K // 8), K // 8), K // 8)],
        h_loc.at[pl.ds(b * (K // 8), K // 8)], sem)
    return ic, hc

  def _oc(cj):
    return pltpu.make_async_copy(
        out_loc,
        out_hbm.at[pl.ds(pl.multiple_of(cj * (K // 8), K // 8), K // 8)],
        osem)

  def _do_chunk(cj, b, carry):
      win, started = carry
      ic, hc = _io(cj, b)
      ic.wait()
      hc.wait()

      @pl.when(cj + NW < NCHUNK)
      def _():
        ic2, hc2 = _io(cj + NW, 1 - b)
        ic2.start()
        hc2.start()

      o = b * K
      ro = b * (K // 8)

      def _group(gi, win):
        g = gi * 16
        sv = ids_loc[pl.ds(g + o, 16)]

        def _gfast(win):
          sid = sv[0]
          win = lax.cond(sid - win >= WAB,
                         lambda w: _reload(sid, w), lambda w: w, win)
          lid = sid - win
          a = ab_win[lid, 0:16]
          b = ab_win[lid, 16:32]
          rw = g >> 3
          for u in range(16):
            cs = (u & 7) * 16
            row = h_loc[ro + rw + (u >> 3), cs:cs + 16]
            out_loc[rw + (u >> 3), cs:cs + 16] = jnp.maximum(row * a + b, 0.0)
          return win

        def _gslow(win):
          for u in range(16):
            sid = sv[u]
            win = lax.cond(sid - win >= WAB,
                           lambda w: _reload(sid, w), lambda w: w, win)
            _apply(g + u, sid, ro, win)
          return win

        return lax.cond(sv[0] == sv[15], _gfast, _gslow, win)

      win = lax.fori_loop(0, K // 16, _group, win)
      _oc(cj).start()
      _oc(cj).wait()
      return (win, jnp.int32(1))

  def _pair(t, carry):
    for b in range(2):
      cj = (2 * t + b) * NW + wid
      carry = lax.cond(cj < NCHUNK,
                       lambda cr, cj=cj, b=b: _do_chunk(cj, b, cr),
                       lambda cr: cr, carry)
    return carry

  cj0 = wid

  @pl.when(cj0 < NCHUNK)
  def _():
    ic, hc = _io(cj0, 0)
    ic.start()
    hc.start()
  nt = (NCHUNK + NW - 1) // NW
  win, started = lax.fori_loop(0, (nt + 1) // 2, _pair,
                               (jnp.int32(-4 * WAB), jnp.int32(0)))


_sc2 = functools.partial(
    pl.kernel,
    out_type=jax.ShapeDtypeStruct((N // 8, 128), jnp.float32),
    mesh=_mesh,
    scratch_types=[
        pltpu.VMEM((2 * K,), jnp.int32),
        pltpu.VMEM((2 * (K // 8), 128), jnp.float32),
        pltpu.VMEM((WAB, 32), jnp.float32),
        pltpu.VMEM((K // 8, 128), jnp.float32),
        pltpu.SemaphoreType.DMA,
        pltpu.SemaphoreType.DMA,
    ],
)


def _run_sc2(h2d, ids32, ab):
  return _sc2(_sc2_body)(h2d, ids32, ab)


def kernel(x, origin_feat, latent_point_batch, W_fc, b_fc, W_mu, b_mu,
           W_sig, b_sig):
  ids32 = latent_point_batch.astype(jnp.int32)
  ids3d_tc = ids32.reshape(N // 12800, 1, 12800)
  x2d = x.reshape(N // 8, 128)
  bi = jnp.arange(128)[:, None] // 16
  bj = jnp.arange(128)[None, :] // 16
  w_bd = jnp.where(bi == bj, jnp.tile(W_fc.T, (8, 8)), 0.0).astype(jnp.float32)
  b_bd = jnp.tile(b_fc, 8).reshape(1, 128)
  h2d, splits = _compute_h(x2d, w_bd, b_bd, ids3d_tc)
  stats2d, cnts = _run_sc1(h2d, ids32, splits.reshape(128))
  stats = stats2d.reshape(NSEGP, 32)[:NSEG]
  ab = _compute_ab(stats, cnts[:NSEG].reshape(NSEG, 1), origin_feat,
                   W_mu, b_mu.reshape(1, OUTC), W_sig, b_sig.reshape(1, OUTC))
  return _run_sc2(h2d, ids32, ab).reshape(N, OUTC)


# speedup vs baseline: 11.0411x; 1.0352x over previous
import functools

import jax
import jax.numpy as jnp
from jax import lax
from jax.experimental import pallas as pl
from jax.experimental.pallas import tpu as pltpu
from jax.experimental.pallas import tpu_sc as plsc

N = 3200000
NSEG = 100000
INC = 16
OUTC = 16
FEATC = 32

NTILE = 16
NCORE = 2
K = 512

_mesh = plsc.VectorSubcoreMesh(core_axis_name="c", subcore_axis_name="s")


def _fc_body(x_ref, wbd_ref, bbd_ref, o_ref):
  h = lax.dot_general(x_ref[...], wbd_ref[...], (((1,), (0,)), ((), ())),
                      preferred_element_type=jnp.float32)
  o_ref[...] = h + bbd_ref[...]


def _compute_h(x2d, w_bd, b_bd):
  blk = 12800
  grid = (N // blk,)
  return pl.pallas_call(
      _fc_body,
      grid=grid,
      in_specs=[
          pl.BlockSpec((blk // 8, 128), lambda i: (i, 0)),
          pl.BlockSpec((128, 128), lambda i: (0, 0)),
          pl.BlockSpec((1, 128), lambda i: (0, 0)),
      ],
      out_specs=pl.BlockSpec((blk // 8, 128), lambda i: (i, 0)),
      out_shape=jax.ShapeDtypeStruct((N // 8, 128), jnp.float32),
  )(x2d, w_bd, b_bd)


SPAN = 3136
NSEGP = 32 * SPAN


def _sc1_body(h_hbm, ids_hbm, stats_hbm, cnts_hbm,
              ids_loc, h_loc, probe, win_d, cnt_win, acc_v, acc2_v, sem):
  c = lax.axis_index("c")
  s = lax.axis_index("s")
  wid = c * NTILE + s
  zero16 = jnp.zeros((16,), jnp.float32)
  zero16i = jnp.zeros((16,), jnp.int32)
  iota16 = lax.iota(jnp.int32, 16)

  def _lane_x(v, lane):
    idx = jnp.where(iota16 == 0, lane, 0)
    return jnp.take_along_axis(v, idx, axis=0)[0]

  def _zw(i, _):
    for u in range(8):
      win_d[i, pl.ds(u * 16, 16)] = zero16
    return 0
  lax.fori_loop(0, SPAN // 4, _zw, 0)

  def _zc(i, _):
    cnt_win[pl.ds(i * 16, 16)] = zero16i
    return 0
  lax.fori_loop(0, SPAN // 16, _zc, 0)

  def _bs(_, st):
    lo0, hi0, lo1, hi1 = st
    mid0 = lax.div(lo0 + hi0, jnp.int32(2))
    base0 = pl.multiple_of(jnp.minimum(mid0 & ~7, N - 16), 8)
    pltpu.sync_copy(ids_hbm.at[pl.ds(base0, 16)], probe)
    w0 = _lane_x(probe[0:16], mid0 - base0)
    mid1 = lax.div(lo1 + hi1, jnp.int32(2))
    base1 = pl.multiple_of(jnp.minimum(mid1 & ~7, N - 16), 8)
    pltpu.sync_copy(ids_hbm.at[pl.ds(base1, 16)], probe)
    w1 = _lane_x(probe[0:16], mid1 - base1)
    t0 = w0 < wid * SPAN
    t1 = w1 < (wid + 1) * SPAN
    return (jnp.where(t0, mid0 + 1, lo0), jnp.where(t0, hi0, mid0),
            jnp.where(t1, mid1 + 1, lo1), jnp.where(t1, hi1, mid1))

  lo_t, _, hi_t, _ = lax.fori_loop(
      0, 22, _bs, (jnp.int32(0), jnp.int32(N), jnp.int32(0), jnp.int32(N)))
  seg_lo = wid * SPAN

  def _emit(cur_seg, cnt):
    lid = jnp.clip(cur_seg - seg_lo, 0, SPAN - 1)
    r = lid >> 2
    col = pl.multiple_of((lid & 3) * 32, 32)
    win_d[r, pl.ds(col, 16)] = acc_v[...]
    win_d[r, pl.ds(col + 16, 16)] = acc2_v[...]
    cb = pl.multiple_of((lid >> 4) << 4, 16)
    cv = cnt_win[pl.ds(cb, 16)]
    cnt_win[pl.ds(cb, 16)] = jnp.where(iota16 == lid - cb, cnt, cv)

  def _step(sid, row, pc):
    def _new(pc):
      cur_seg, cnt = pc
      _emit(cur_seg, cnt)
      acc_v[...] = row
      acc2_v[...] = row * row
      return (sid, jnp.int32(1))

    def _same(pc):
      cur_seg, cnt = pc
      plsc.addupdate(acc_v.at[...], row)
      plsc.addupdate(acc2_v.at[...], row * row)
      return (sid, cnt + 1)

    return lax.cond(sid != pc[0], _new, _same, pc)

  def _point(i, o, ro, pc):
    g = pl.multiple_of((i >> 4) << 4, 16)
    sv = ids_loc[pl.ds(g + o, 16)]
    sid = _lane_x(sv, i - g)
    col = pl.multiple_of((i & 7) * 16, 16)
    return _step(sid, h_loc[ro + (i >> 3), pl.ds(col, 16)], pc)

  def _group(gi, o, ro, pc):
    g = gi * 16
    sv = ids_loc[pl.ds(g + o, 16)]
    rw = (g >> 3) + ro

    def _fast(pc):
      rows = [h_loc[rw + (u >> 3), (u & 7) * 16:(u & 7) * 16 + 16]
              for u in range(16)]
      sm = rows[0]
      sq = rows[0] * rows[0]
      for u in range(1, 16):
        sm = sm + rows[u]
        sq = sq + rows[u] * rows[u]

      def _cont(pc):
        cur_seg, cnt = pc
        plsc.addupdate(acc_v.at[...], sm)
        plsc.addupdate(acc2_v.at[...], sq)
        return (cur_seg, cnt + 16)

      def _brk(pc):
        cur_seg, cnt = pc
        _emit(cur_seg, cnt)
        acc_v[...] = sm
        acc2_v[...] = sq
        return (sv[0], jnp.int32(16))

      return lax.cond(sv[0] == pc[0], _cont, _brk, pc)

    def _slow(pc):
      for u in range(16):
        pc = _step(sv[u],
                   h_loc[rw + (u >> 3), (u & 7) * 16:(u & 7) * 16 + 16],
                   pc)
      return pc

    return lax.cond(sv[0] == sv[15], _fast, _slow, pc)

  def _io(cj, b):
    base = pl.multiple_of(cj * K, K)
    ic = pltpu.make_async_copy(ids_hbm.at[pl.ds(base, K)],
                               ids_loc.at[pl.ds(b * K, K)], sem)
    hc = pltpu.make_async_copy(
        h_hbm.at[pl.ds(pl.multiple_of(cj * (K // 8), K // 8), K // 8)],
        h_loc.at[pl.ds(b * (K // 8), K // 8)], sem)
    return ic, hc

  def _issue(cj, b):
    ic, hc = _io(cj, b)
    ic.start()
    hc.start()

  def _proc(cj, b, carry):
    ic, hc = _io(cj, b)
    ic.wait()
    hc.wait()

    @pl.when(cj + 1 < nc1)
    def _():
      _issue(cj + 1, 1 - b)

    base = pl.multiple_of(cj * K, K)
    o = b * K
    ro = b * (K // 8)
    i0 = jnp.maximum(lo_t - base, 0)
    i1 = jnp.minimum(hi_t - base, K)
    a = jnp.minimum((i0 + 15) & ~15, i1)
    bb = jnp.maximum(i1 & ~15, a)
    carry = lax.fori_loop(i0, a, lambda i, pc: _point(i, o, ro, pc), carry)
    carry = lax.fori_loop(a >> 4, bb >> 4,
                          lambda gi, pc: _group(gi, o, ro, pc), carry)
    carry = lax.fori_loop(bb, i1, lambda i, pc: _point(i, o, ro, pc), carry)
    return carry

  def _pair(t, carry):
    for b in range(2):
      cj = nc0 + 2 * t + b
      carry = lax.cond(cj < nc1,
                       lambda pc, cj=cj, b=b: _proc(cj, b, pc),
                       lambda pc: pc, carry)
    return carry

  acc_v[...] = zero16
  acc2_v[...] = zero16
  init = (jnp.int32(-1), jnp.int32(0))
  nc0 = lax.div(lo_t, jnp.int32(K))
  nc1 = lax.div(hi_t + K - 1, jnp.int32(K))

  @pl.when(nc1 > nc0)
  def _():
    _issue(nc0, 0)
  cur_seg, cnt = lax.fori_loop(0, lax.div(nc1 - nc0 + 1, jnp.int32(2)),
                               _pair, init)
  _emit(jnp.where(cnt > 0, cur_seg, seg_lo), cnt)

  pltpu.sync_copy(win_d, stats_hbm.at[pl.ds(wid * (SPAN // 4), SPAN // 4)])
  pltpu.sync_copy(cnt_win, cnts_hbm.at[pl.ds(wid * SPAN, SPAN)])


_sc1 = functools.partial(
    pl.kernel,
    out_type=(jax.ShapeDtypeStruct((NSEGP // 4, 128), jnp.float32),
              jax.ShapeDtypeStruct((NSEGP,), jnp.int32)),
    mesh=_mesh,
    scratch_types=[
        pltpu.VMEM((2 * K,), jnp.int32),
        pltpu.VMEM((2 * (K // 8), 128), jnp.float32),
        pltpu.VMEM((16,), jnp.int32),
        pltpu.VMEM((SPAN // 4, 128), jnp.float32),
        pltpu.VMEM((SPAN,), jnp.int32),
        pltpu.VMEM((16,), jnp.float32),
        pltpu.VMEM((16,), jnp.float32),
        pltpu.SemaphoreType.DMA,
    ],
)


def _run_sc1(h2d, ids32):
  return _sc1(_sc1_body)(h2d, ids32)


def _ab_body(st_ref, ct_ref, of_ref, wmu_ref, bmu_ref, wsig_ref, bsig_ref,
             ab_ref):
  sums = st_ref[:, 0:16]
  sqs = st_ref[:, 16:32]
  cnt = jnp.maximum(ct_ref[...].astype(jnp.float32), 1.0)
  mean = sums / cnt
  var = jnp.maximum(sqs / cnt - mean * mean, 0.0)
  rstd = lax.rsqrt(var + 1e-14)
  of = of_ref[...]
  mu = lax.dot_general(of, wmu_ref[...], (((1,), (1,)), ((), ())),
                       preferred_element_type=jnp.float32) + bmu_ref[...]
  sig = lax.dot_general(of, wsig_ref[...], (((1,), (1,)), ((), ())),
                        preferred_element_type=jnp.float32) + bsig_ref[...]
  a = sig * rstd
  b = mu - mean * a
  ab_ref[:, 0:16] = a
  ab_ref[:, 16:32] = b


def _compute_ab(stats, cnts, origin_feat, w_mu, b_mu, w_sig, b_sig):
  blk = 4000
  grid = (NSEG // blk,)
  return pl.pallas_call(
      _ab_body,
      grid=grid,
      in_specs=[
          pl.BlockSpec((blk, 32), lambda i: (i, 0)),
          pl.BlockSpec((blk, 1), lambda i: (i, 0)),
          pl.BlockSpec((blk, FEATC), lambda i: (i, 0)),
          pl.BlockSpec((OUTC, FEATC), lambda i: (0, 0)),
          pl.BlockSpec((1, OUTC), lambda i: (0, 0)),
          pl.BlockSpec((OUTC, FEATC), lambda i: (0, 0)),
          pl.BlockSpec((1, OUTC), lambda i: (0, 0)),
      ],
      out_specs=pl.BlockSpec((blk, 32), lambda i: (i, 0)),
      out_shape=jax.ShapeDtypeStruct((NSEG, 32), jnp.float32),
  )(stats, cnts, origin_feat, w_mu, b_mu, w_sig, b_sig)


NCHUNK = N // K
NW = NCORE * NTILE
WAB = 256


def _sc2_body(h_hbm, ids_hbm, ab_hbm, out_hbm,
              ids_loc, h_loc, ab_win, out_loc, sem, osem):
  c = lax.axis_index("c")
  s = lax.axis_index("s")
  wid = s * NCORE + c
  iota16 = lax.iota(jnp.int32, 16)

  def _reload(sid, win):
    nw = pl.multiple_of(jnp.minimum(sid & ~7, NSEG - WAB), 8)
    pltpu.sync_copy(ab_hbm.at[pl.ds(nw, WAB)], ab_win)
    return nw

  def _apply(pt, sid, ro, win):
    col = pl.multiple_of((pt & 7) * 16, 16)
    row = h_loc[ro + (pt >> 3), pl.ds(col, 16)]
    lid = sid - win
    a = ab_win[lid, 0:16]
    b = ab_win[lid, 16:32]
    out_loc[pt >> 3, pl.ds(col, 16)] = jnp.maximum(row * a + b, 0.0)

  def _io(cj, b):
    base = pl.multiple_of(cj * K, K)
    ic = pltpu.make_async_copy(ids_hbm.at[pl.ds(base, K)],
                               ids_loc.at[pl.ds(b * K, K)], sem)
    hc = pltpu.make_async_copy(
        h_hbm.at[pl.ds(pl.multiple_of(cj * (K // 8), K // 8), K // 8)],
        h_loc.at[pl.ds(b * (K // 8), K // 8)], sem)
    return ic, hc

  def _oc(cj):
    return pltpu.make_async_copy(
        out_loc,
        out_hbm.at[pl.ds(pl.multiple_of(cj * (K // 8), K // 8), K // 8)],
        osem)

  def _do_chunk(cj, b, carry):
      win, started = carry
      ic, hc = _io(cj, b)
      ic.wait()
      hc.wait()

      @pl.when(cj + NW < NCHUNK)
      def _():
        ic2, hc2 = _io(cj + NW, 1 - b)
        ic2.start()
        hc2.start()

      o = b * K
      ro = b * (K // 8)

      def _group(gi, win):
        g = gi * 16
        sv = ids_loc[pl.ds(g + o, 16)]

        def _gfast(win):
          sid = sv[0]
          win = lax.cond(sid - win >= WAB,
                         lambda w: _reload(sid, w), lambda w: w, win)
          lid = sid - win
          a = ab_win[lid, 0:16]
          b = ab_win[lid, 16:32]
          rw = g >> 3
          for u in range(16):
            cs = (u & 7) * 16
            row = h_loc[ro + rw + (u >> 3), cs:cs + 16]
            out_loc[rw + (u >> 3), cs:cs + 16] = jnp.maximum(row * a + b, 0.0)
          return win

        def _gslow(win):
          for u in range(16):
            sid = sv[u]
            win = lax.cond(sid - win >= WAB,
                           lambda w: _reload(sid, w), lambda w: w, win)
            _apply(g + u, sid, ro, win)
          return win

        return lax.cond(sv[0] == sv[15], _gfast, _gslow, win)

      win = lax.fori_loop(0, K // 16, _group, win)
      _oc(cj).start()
      _oc(cj).wait()
      return (win, jnp.int32(1))

  def _pair(t, carry):
    for b in range(2):
      cj = (2 * t + b) * NW + wid
      carry = lax.cond(cj < NCHUNK,
                       lambda cr, cj=cj, b=b: _do_chunk(cj, b, cr),
                       lambda cr: cr, carry)
    return carry

  cj0 = wid

  @pl.when(cj0 < NCHUNK)
  def _():
    ic, hc = _io(cj0, 0)
    ic.start()
    hc.start()
  nt = (NCHUNK + NW - 1) // NW
  win, started = lax.fori_loop(0, (nt + 1) // 2, _pair,
                               (jnp.int32(-4 * WAB), jnp.int32(0)))


_sc2 = functools.partial(
    pl.kernel,
    out_type=jax.ShapeDtypeStruct((N // 8, 128), jnp.float32),
    mesh=_mesh,
    scratch_types=[
        pltpu.VMEM((2 * K,), jnp.int32),
        pltpu.VMEM((2 * (K // 8), 128), jnp.float32),
        pltpu.VMEM((WAB, 32), jnp.float32),
        pltpu.VMEM((K // 8, 128), jnp.float32),
        pltpu.SemaphoreType.DMA,
        pltpu.SemaphoreType.DMA,
    ],
)


def _run_sc2(h2d, ids32, ab):
  return _sc2(_sc2_body)(h2d, ids32, ab)


def kernel(x, origin_feat, latent_point_batch, W_fc, b_fc, W_mu, b_mu,
           W_sig, b_sig):
  ids32 = latent_point_batch.astype(jnp.int32)
  x2d = x.reshape(N // 8, 128)
  bi = jnp.arange(128)[:, None] // 16
  bj = jnp.arange(128)[None, :] // 16
  w_bd = jnp.where(bi == bj, jnp.tile(W_fc.T, (8, 8)), 0.0).astype(jnp.float32)
  b_bd = jnp.tile(b_fc, 8).reshape(1, 128)
  h2d = _compute_h(x2d, w_bd, b_bd)
  stats2d, cnts = _run_sc1(h2d, ids32)
  stats = stats2d.reshape(NSEGP, 32)[:NSEG]
  ab = _compute_ab(stats, cnts[:NSEG].reshape(NSEG, 1), origin_feat,
                   W_mu, b_mu.reshape(1, OUTC), W_sig, b_sig.reshape(1, OUTC))
  return _run_sc2(h2d, ids32, ab).reshape(N, OUTC)
